# baseline (speedup 1.0000x reference)
import functools

import jax
import jax.numpy as jnp
from jax import lax
from jax.experimental import pallas as pl
from jax.experimental.pallas import tpu as pltpu
from jax.experimental.pallas import tpu_sc as plsc

N_NODES = 10000
N_EDGES = 320000
D_FEAT = 128
D_EDGE = 16

NC, NS = 2, 16
NW = NC * NS
BATCH = 128
NB = N_EDGES // BATCH
CH = 8
NCHUNK = 312
NSLOT = 10
TAIL_B = NB - NCHUNK * CH
ACC_ROWS = 10240
ROWS_PER_TILE = ACC_ROWS // NS
PACK_PER_TILE = ROWS_PER_TILE // 8

_mesh = plsc.VectorSubcoreMesh(core_axis_name="c", subcore_axis_name="s")


@functools.partial(
    pl.kernel,
    out_type=jax.ShapeDtypeStruct((NC * ACC_ROWS // 8, D_FEAT), jnp.float32),
    mesh=_mesh,
    scratch_types=[
        pltpu.VMEM((2, CH, 2, BATCH), jnp.int32),
        pltpu.VMEM((2, CH, D_EDGE, BATCH), jnp.float32),
        pltpu.VMEM((2, CH * BATCH, D_EDGE), jnp.float32),
        pltpu.VMEM((TAIL_B, 2, BATCH), jnp.int32),
        pltpu.VMEM((ROWS_PER_TILE, D_EDGE), jnp.float32),
        pltpu.VMEM((PACK_PER_TILE, D_FEAT), jnp.float32),
        pltpu.VMEM_SHARED((ACC_ROWS, D_EDGE), jnp.float32),
        pltpu.SemaphoreType.DMA,
        pltpu.SemaphoreType.DMA,
        pltpu.SemaphoreType.DMA,
        pltpu.SemaphoreType.DMA,
        pltpu.SemaphoreType.DMA,
    ],
    compiler_params=pltpu.CompilerParams(
        use_tc_tiling_on_sc=False, needs_layout_passes=False),
)
def _sc_segment_sum(idx_hbm, attr_hbm, zeros_hbm,
                    out_hbm, idx_v, a_v, edge_v, idxt_v,
                    stripe_v, pack_v, acc_sh,
                    sem_z, sem_l0, sem_l1, sem_s0, sem_s1):
    c = lax.axis_index("c")
    s = lax.axis_index("s")
    wid = s * NC + c
    sem_l = (sem_l0, sem_l1)
    sem_s = (sem_s0, sem_s1)

    my_rows = pl.ds(s * ROWS_PER_TILE, ROWS_PER_TILE)
    zero_d = pltpu.async_copy(zeros_hbm, acc_sh.at[my_rows], sem_z)

    def issue_loads(t, b):
        chunk = wid + NW * t
        row0 = pl.ds(chunk * CH, CH)
        return (
            pltpu.async_copy(idx_hbm.at[row0], idx_v.at[b], sem_l[b]),
            pltpu.async_copy(attr_hbm.at[0, row0],
                             a_v.at[b, :, pl.ds(0, 8)], sem_l[b]),
            pltpu.async_copy(attr_hbm.at[1, row0],
                             a_v.at[b, :, pl.ds(8, 8)], sem_l[b]),
        )

    lane = lax.iota(jnp.int32, 16)

    def transpose_chunk(b, n_batches):
        def body(m, carry):
            jb = m // 8
            e0 = (m % 8) * 16
            rows = jb * BATCH + e0 + lane
            for f in range(D_EDGE):
                fcol = jnp.full((16,), f, jnp.int32)
                plsc.store_scatter(edge_v.at[b], [rows, fcol],
                                   a_v[b, jb, f, pl.ds(e0, 16)])
            return carry

        lax.fori_loop(0, n_batches * 8, body, 0, unroll=4)

    def issue_scatters(b):
        return [
            pltpu.async_copy(edge_v.at[b, pl.ds(j * BATCH, BATCH)],
                             acc_sh.at[idx_v.at[b, j, 1]], sem_s[b], add=True)
            for j in range(CH)
        ]

    def drain(descs):
        for d in descs:
            d.wait()

    loads = [None, None]
    scats = [None, None]
    loads[0] = issue_loads(0, 0)
    zero_d.wait()
    plsc.subcore_barrier()

    for t in range(NSLOT - 1):
        b = t & 1
        nb = b ^ 1
        drain(loads[b])
        if scats[nb] is not None:
            drain(scats[nb])
            scats[nb] = None
        if t + 1 < NSLOT - 1:
            loads[nb] = issue_loads(t + 1, nb)
        transpose_chunk(b, CH)
        scats[b] = issue_scatters(b)

    drain(scats[0])

    @pl.when(wid < NCHUNK - NW * (NSLOT - 1))
    def _slot9():
        l9 = issue_loads(NSLOT - 1, 1)
        drain(l9)
        transpose_chunk(1, CH)
        drain(issue_scatters(1))

    @pl.when(wid == NW - 1)
    def _tail():
        row0 = pl.ds(NCHUNK * CH, TAIL_B)
        pltpu.sync_copy(idx_hbm.at[row0], idxt_v)
        pltpu.sync_copy(attr_hbm.at[0, row0], a_v.at[1, pl.ds(0, TAIL_B),
                                                     pl.ds(0, 8)])
        pltpu.sync_copy(attr_hbm.at[1, row0], a_v.at[1, pl.ds(0, TAIL_B),
                                                     pl.ds(8, 8)])
        transpose_chunk(1, TAIL_B)
        tds = [
            pltpu.async_copy(edge_v.at[1, pl.ds(j * BATCH, BATCH)],
                             acc_sh.at[idxt_v.at[j, 1]], sem_s[1], add=True)
            for j in range(TAIL_B)
        ]
        drain(tds)

    plsc.subcore_barrier()
    pltpu.sync_copy(acc_sh.at[my_rows], stripe_v)

    def pack_body(r, carry):
        for a in range(8):
            pack_v[r, pl.ds(a * D_EDGE, D_EDGE)] = stripe_v[r * 8 + a, :]
        return carry

    lax.fori_loop(0, PACK_PER_TILE, pack_body, 0)
    pltpu.sync_copy(
        pack_v,
        out_hbm.at[pl.ds(c * (ACC_ROWS // 8) + s * PACK_PER_TILE,
                         PACK_PER_TILE)],
    )


BLK = 1024


def _tc_mlp_body(x_ref, p0_ref, p1_ref, w1a_ref, w1b_ref, b1_ref, w2_ref,
                 b2_ref, g_ref, bt_ref, o_ref):
    x = x_ref[...]
    pp = p0_ref[...] + p1_ref[...]
    wrep = jnp.concatenate([jnp.concatenate([w1b_ref[...]] * 8, axis=0)] * 8,
                           axis=1)
    krow = jax.lax.broadcasted_iota(jnp.int32, (D_FEAT, 8 * D_FEAT), 0)
    kcol = jax.lax.broadcasted_iota(jnp.int32, (D_FEAT, 8 * D_FEAT), 1)
    wbig = jnp.where((krow // D_EDGE) == (kcol // D_FEAT), wrep, 0.0)
    hagg = jnp.dot(pp, wbig, preferred_element_type=jnp.float32)
    hagg = hagg.reshape(BLK // 8, 8, D_FEAT).reshape(BLK, D_FEAT)
    h = jnp.dot(x, w1a_ref[...], preferred_element_type=jnp.float32)
    h = h + hagg + b1_ref[...]
    h = h * jax.nn.sigmoid(h)
    h = jnp.dot(h, w2_ref[...], preferred_element_type=jnp.float32) + b2_ref[...]
    mu = jnp.mean(h, axis=-1, keepdims=True)
    d = h - mu
    var = jnp.mean(d * d, axis=-1, keepdims=True)
    hn = d * lax.rsqrt(var + 1e-5)
    o_ref[...] = x + hn * g_ref[...] + bt_ref[...]


def _full_spec(nr, nc):
    return pl.BlockSpec((nr, nc), lambda i: (0, 0))


_tc_mlp = pl.pallas_call(
    _tc_mlp_body,
    grid=((N_NODES + BLK - 1) // BLK,),
    in_specs=[
        pl.BlockSpec((BLK, D_FEAT), lambda i: (i, 0)),
        pl.BlockSpec((BLK // 8, D_FEAT), lambda i: (i, 0)),
        pl.BlockSpec((BLK // 8, D_FEAT),
                     lambda i: (i + ACC_ROWS // BLK, 0)),
        _full_spec(D_FEAT, D_FEAT),
        _full_spec(D_EDGE, D_FEAT),
        _full_spec(1, D_FEAT),
        _full_spec(D_FEAT, D_FEAT),
        _full_spec(1, D_FEAT),
        _full_spec(1, D_FEAT),
        _full_spec(1, D_FEAT),
    ],
    out_specs=pl.BlockSpec((BLK, D_FEAT), lambda i: (i, 0)),
    out_shape=jax.ShapeDtypeStruct((N_NODES, D_FEAT), jnp.float32),
)


def kernel(x, edge_index, edge_attr, W1, b1, W2, b2, ln_gamma, ln_beta):
    idx3 = edge_index.astype(jnp.int32).reshape(2, NB, BATCH).transpose(1, 0, 2)
    z = edge_attr.T.reshape(2, 8, NB, BATCH).transpose(0, 2, 1, 3)
    zeros = jnp.zeros((ROWS_PER_TILE, D_EDGE), jnp.float32)
    partial = _sc_segment_sum(idx3, z, zeros)
    return _tc_mlp(
        x, partial, partial,
        W1[:D_FEAT], W1[D_FEAT:],
        b1.reshape(1, D_FEAT), W2, b2.reshape(1, D_FEAT),
        ln_gamma.reshape(1, D_FEAT), ln_beta.reshape(1, D_FEAT),
    )


# speedup vs baseline: 1.0337x; 1.0337x over previous
import functools

import jax
import jax.numpy as jnp
from jax import lax
from jax.experimental import pallas as pl
from jax.experimental.pallas import tpu as pltpu
from jax.experimental.pallas import tpu_sc as plsc

N_NODES = 10000
N_EDGES = 320000
D_FEAT = 128
D_EDGE = 16

NC, NS = 2, 16
NW = NC * NS
BATCH = 128
NB = N_EDGES // BATCH
CH = 8
NCHUNK = 312
NSLOT = 10
TAIL_B = NB - NCHUNK * CH
ACC_ROWS = 10240
ROWS_PER_TILE = ACC_ROWS // NS
PACK_PER_TILE = ROWS_PER_TILE // 8

_mesh = plsc.VectorSubcoreMesh(core_axis_name="c", subcore_axis_name="s")


@functools.partial(
    pl.kernel,
    out_type=jax.ShapeDtypeStruct((NC * ACC_ROWS // 8, D_FEAT), jnp.float32),
    mesh=_mesh,
    scratch_types=[
        pltpu.VMEM((2, CH, 2, BATCH), jnp.int32),
        pltpu.VMEM((2, CH, D_EDGE, BATCH), jnp.float32),
        pltpu.VMEM((2, CH * BATCH, D_EDGE), jnp.float32),
        pltpu.VMEM((TAIL_B, 2, BATCH), jnp.int32),
        pltpu.VMEM((ROWS_PER_TILE, D_EDGE), jnp.float32),
        pltpu.VMEM((PACK_PER_TILE, D_FEAT), jnp.float32),
        pltpu.VMEM_SHARED((ACC_ROWS, D_EDGE), jnp.float32),
        pltpu.SemaphoreType.DMA,
        pltpu.SemaphoreType.DMA,
        pltpu.SemaphoreType.DMA,
        pltpu.SemaphoreType.DMA,
        pltpu.SemaphoreType.DMA,
    ],
    compiler_params=pltpu.CompilerParams(
        use_tc_tiling_on_sc=False, needs_layout_passes=False),
)
def _sc_segment_sum(idx_hbm, attr_hbm, zeros_hbm,
                    out_hbm, idx_v, a_v, edge_v, idxt_v,
                    stripe_v, pack_v, acc_sh,
                    sem_z, sem_l0, sem_l1, sem_s0, sem_s1):
    c = lax.axis_index("c")
    s = lax.axis_index("s")
    wid = s * NC + c
    sem_l = (sem_l0, sem_l1)
    sem_s = (sem_s0, sem_s1)

    my_rows = pl.ds(s * ROWS_PER_TILE, ROWS_PER_TILE)
    zero_d = pltpu.async_copy(zeros_hbm, acc_sh.at[my_rows], sem_z)

    def issue_loads(t, b):
        chunk = wid + NW * t
        row0 = pl.ds(chunk * CH, CH)
        return (
            pltpu.async_copy(idx_hbm.at[row0], idx_v.at[b], sem_l[b]),
            pltpu.async_copy(attr_hbm.at[0, row0],
                             a_v.at[b, :, pl.ds(0, 8)], sem_l[b]),
            pltpu.async_copy(attr_hbm.at[1, row0],
                             a_v.at[b, :, pl.ds(8, 8)], sem_l[b]),
        )

    lane = lax.iota(jnp.int32, 16)
    fcols = [jnp.full((16,), f, jnp.int32) for f in range(D_EDGE)]

    def transpose_chunk(b, n_batches):
        def body(m, carry):
            jb = m // 8
            e0 = (m % 8) * 16
            rows = jb * BATCH + e0 + lane
            for f in range(D_EDGE):
                plsc.store_scatter(edge_v.at[b], [rows, fcols[f]],
                                   a_v[b, jb, f, pl.ds(e0, 16)])
            return carry

        lax.fori_loop(0, n_batches * 8, body, 0, unroll=2)

    def issue_scatters(b):
        return [
            pltpu.async_copy(edge_v.at[b, pl.ds(j * BATCH, BATCH)],
                             acc_sh.at[idx_v.at[b, j, 1]], sem_s[b], add=True)
            for j in range(CH)
        ]

    def drain(descs):
        for d in descs:
            d.wait()

    loads = [None, None]
    scats = [None, None]
    loads[0] = issue_loads(0, 0)
    zero_d.wait()
    plsc.subcore_barrier()

    for t in range(NSLOT - 1):
        b = t & 1
        nb = b ^ 1
        drain(loads[b])
        if scats[nb] is not None:
            drain(scats[nb])
            scats[nb] = None
        if t + 1 < NSLOT - 1:
            loads[nb] = issue_loads(t + 1, nb)
        transpose_chunk(b, CH)
        scats[b] = issue_scatters(b)

    drain(scats[0])

    @pl.when(wid < NCHUNK - NW * (NSLOT - 1))
    def _slot9():
        l9 = issue_loads(NSLOT - 1, 1)
        drain(l9)
        transpose_chunk(1, CH)
        drain(issue_scatters(1))

    @pl.when(wid == NW - 1)
    def _tail():
        row0 = pl.ds(NCHUNK * CH, TAIL_B)
        pltpu.sync_copy(idx_hbm.at[row0], idxt_v)
        pltpu.sync_copy(attr_hbm.at[0, row0], a_v.at[1, pl.ds(0, TAIL_B),
                                                     pl.ds(0, 8)])
        pltpu.sync_copy(attr_hbm.at[1, row0], a_v.at[1, pl.ds(0, TAIL_B),
                                                     pl.ds(8, 8)])
        transpose_chunk(1, TAIL_B)
        tds = [
            pltpu.async_copy(edge_v.at[1, pl.ds(j * BATCH, BATCH)],
                             acc_sh.at[idxt_v.at[j, 1]], sem_s[1], add=True)
            for j in range(TAIL_B)
        ]
        drain(tds)

    plsc.subcore_barrier()
    pltpu.sync_copy(acc_sh.at[my_rows], stripe_v)

    def pack_body(r, carry):
        for a in range(8):
            pack_v[r, pl.ds(a * D_EDGE, D_EDGE)] = stripe_v[r * 8 + a, :]
        return carry

    lax.fori_loop(0, PACK_PER_TILE, pack_body, 0)
    pltpu.sync_copy(
        pack_v,
        out_hbm.at[pl.ds(c * (ACC_ROWS // 8) + s * PACK_PER_TILE,
                         PACK_PER_TILE)],
    )


BLK = 1024


def _tc_mlp_body(x_ref, p0_ref, p1_ref, w1a_ref, w1b_ref, b1_ref, w2_ref,
                 b2_ref, g_ref, bt_ref, o_ref):
    x = x_ref[...]
    pp = p0_ref[...] + p1_ref[...]
    wrep = jnp.concatenate([jnp.concatenate([w1b_ref[...]] * 8, axis=0)] * 8,
                           axis=1)
    krow = jax.lax.broadcasted_iota(jnp.int32, (D_FEAT, 8 * D_FEAT), 0)
    kcol = jax.lax.broadcasted_iota(jnp.int32, (D_FEAT, 8 * D_FEAT), 1)
    wbig = jnp.where((krow // D_EDGE) == (kcol // D_FEAT), wrep, 0.0)
    hagg = jnp.dot(pp, wbig, preferred_element_type=jnp.float32)
    hagg = hagg.reshape(BLK // 8, 8, D_FEAT).reshape(BLK, D_FEAT)
    h = jnp.dot(x, w1a_ref[...], preferred_element_type=jnp.float32)
    h = h + hagg + b1_ref[...]
    h = h * jax.nn.sigmoid(h)
    h = jnp.dot(h, w2_ref[...], preferred_element_type=jnp.float32) + b2_ref[...]
    mu = jnp.mean(h, axis=-1, keepdims=True)
    d = h - mu
    var = jnp.mean(d * d, axis=-1, keepdims=True)
    hn = d * lax.rsqrt(var + 1e-5)
    o_ref[...] = x + hn * g_ref[...] + bt_ref[...]


def _full_spec(nr, nc):
    return pl.BlockSpec((nr, nc), lambda i: (0, 0))


_tc_mlp = pl.pallas_call(
    _tc_mlp_body,
    grid=((N_NODES + BLK - 1) // BLK,),
    in_specs=[
        pl.BlockSpec((BLK, D_FEAT), lambda i: (i, 0)),
        pl.BlockSpec((BLK // 8, D_FEAT), lambda i: (i, 0)),
        pl.BlockSpec((BLK // 8, D_FEAT),
                     lambda i: (i + ACC_ROWS // BLK, 0)),
        _full_spec(D_FEAT, D_FEAT),
        _full_spec(D_EDGE, D_FEAT),
        _full_spec(1, D_FEAT),
        _full_spec(D_FEAT, D_FEAT),
        _full_spec(1, D_FEAT),
        _full_spec(1, D_FEAT),
        _full_spec(1, D_FEAT),
    ],
    out_specs=pl.BlockSpec((BLK, D_FEAT), lambda i: (i, 0)),
    out_shape=jax.ShapeDtypeStruct((N_NODES, D_FEAT), jnp.float32),
)


def kernel(x, edge_index, edge_attr, W1, b1, W2, b2, ln_gamma, ln_beta):
    idx3 = edge_index.astype(jnp.int32).reshape(2, NB, BATCH).transpose(1, 0, 2)
    z = edge_attr.T.reshape(2, 8, NB, BATCH).transpose(0, 2, 1, 3)
    zeros = jnp.zeros((ROWS_PER_TILE, D_EDGE), jnp.float32)
    partial = _sc_segment_sum(idx3, z, zeros)
    return _tc_mlp(
        x, partial, partial,
        W1[:D_FEAT], W1[D_FEAT:],
        b1.reshape(1, D_FEAT), W2, b2.reshape(1, D_FEAT),
        ln_gamma.reshape(1, D_FEAT), ln_beta.reshape(1, D_FEAT),
    )


# speedup vs baseline: 1.0699x; 1.0350x over previous
import functools

import jax
import jax.numpy as jnp
from jax import lax
from jax.experimental import pallas as pl
from jax.experimental.pallas import tpu as pltpu
from jax.experimental.pallas import tpu_sc as plsc

N_NODES = 10000
N_EDGES = 320000
D_FEAT = 128
D_EDGE = 16

NC, NS = 2, 16
NW = NC * NS
BATCH = 128
NB = N_EDGES // BATCH
CH = 8
NCHUNK = 312
NSLOT = 10
TAIL_B = NB - NCHUNK * CH
ACC_ROWS = 10240
ROWS_PER_TILE = ACC_ROWS // NS
PACK_PER_TILE = ROWS_PER_TILE // 8

_mesh = plsc.VectorSubcoreMesh(core_axis_name="c", subcore_axis_name="s")


@functools.partial(
    pl.kernel,
    out_type=jax.ShapeDtypeStruct((NC * ACC_ROWS // 8, D_FEAT), jnp.float32),
    mesh=_mesh,
    scratch_types=[
        pltpu.VMEM((2, CH, 2, BATCH), jnp.int32),
        pltpu.VMEM((2, CH, D_EDGE, BATCH), jnp.float32),
        pltpu.VMEM((2, CH * BATCH, D_EDGE), jnp.float32),
        pltpu.VMEM((TAIL_B, 2, BATCH), jnp.int32),
        pltpu.VMEM((ROWS_PER_TILE, D_EDGE), jnp.float32),
        pltpu.VMEM((PACK_PER_TILE, D_FEAT), jnp.float32),
        pltpu.VMEM_SHARED((ACC_ROWS, D_EDGE), jnp.float32),
        pltpu.SemaphoreType.DMA,
        pltpu.SemaphoreType.DMA,
        pltpu.SemaphoreType.DMA,
        pltpu.SemaphoreType.DMA,
        pltpu.SemaphoreType.DMA,
    ],
    compiler_params=pltpu.CompilerParams(
        use_tc_tiling_on_sc=False, needs_layout_passes=False),
)
def _sc_segment_sum(idx_hbm, attr_hbm, zeros_hbm,
                    out_hbm, idx_v, a_v, edge_v, idxt_v,
                    stripe_v, pack_v, acc_sh,
                    sem_z, sem_l0, sem_l1, sem_s0, sem_s1):
    c = lax.axis_index("c")
    s = lax.axis_index("s")
    wid = s * NC + c
    sem_l = (sem_l0, sem_l1)
    sem_s = (sem_s0, sem_s1)

    my_rows = pl.ds(s * ROWS_PER_TILE, ROWS_PER_TILE)
    zero_d = pltpu.async_copy(zeros_hbm, acc_sh.at[my_rows], sem_z)

    def issue_loads(t, b):
        chunk = wid + NW * t
        row0 = pl.ds(chunk * CH, CH)
        return (
            pltpu.async_copy(idx_hbm.at[row0], idx_v.at[b], sem_l[b]),
            pltpu.async_copy(attr_hbm.at[0, row0],
                             a_v.at[b, :, pl.ds(0, 8)], sem_l[b]),
            pltpu.async_copy(attr_hbm.at[1, row0],
                             a_v.at[b, :, pl.ds(8, 8)], sem_l[b]),
        )

    lane = lax.iota(jnp.int32, 16)
    fcols = [jnp.full((16,), f, jnp.int32) for f in range(D_EDGE)]

    def transpose_chunk(b, n_batches):
        def body(m):
            jb = m // 8
            e0 = (m % 8) * 16
            rows = jb * BATCH + e0 + lane
            for f in range(D_EDGE):
                plsc.store_scatter(edge_v.at[b], [rows, fcols[f]],
                                   a_v[b, jb, f, pl.ds(e0, 16)])

        plsc.parallel_loop(0, n_batches * 8, 1, unroll=4)(body)

    def issue_scatters(b):
        return [
            pltpu.async_copy(edge_v.at[b, pl.ds(j * BATCH, BATCH)],
                             acc_sh.at[idx_v.at[b, j, 1]], sem_s[b], add=True)
            for j in range(CH)
        ]

    def drain(descs):
        for d in descs:
            d.wait()

    loads = [None, None]
    scats = [None, None]
    loads[0] = issue_loads(0, 0)
    zero_d.wait()
    plsc.subcore_barrier()

    for t in range(NSLOT - 1):
        b = t & 1
        nb = b ^ 1
        drain(loads[b])
        if scats[nb] is not None:
            drain(scats[nb])
            scats[nb] = None
        if t + 1 < NSLOT - 1:
            loads[nb] = issue_loads(t + 1, nb)
        transpose_chunk(b, CH)
        scats[b] = issue_scatters(b)

    drain(scats[0])

    @pl.when(wid < NCHUNK - NW * (NSLOT - 1))
    def _slot9():
        l9 = issue_loads(NSLOT - 1, 1)
        drain(l9)
        transpose_chunk(1, CH)
        drain(issue_scatters(1))

    @pl.when(wid == NW - 1)
    def _tail():
        row0 = pl.ds(NCHUNK * CH, TAIL_B)
        pltpu.sync_copy(idx_hbm.at[row0], idxt_v)
        pltpu.sync_copy(attr_hbm.at[0, row0], a_v.at[1, pl.ds(0, TAIL_B),
                                                     pl.ds(0, 8)])
        pltpu.sync_copy(attr_hbm.at[1, row0], a_v.at[1, pl.ds(0, TAIL_B),
                                                     pl.ds(8, 8)])
        transpose_chunk(1, TAIL_B)
        tds = [
            pltpu.async_copy(edge_v.at[1, pl.ds(j * BATCH, BATCH)],
                             acc_sh.at[idxt_v.at[j, 1]], sem_s[1], add=True)
            for j in range(TAIL_B)
        ]
        drain(tds)

    plsc.subcore_barrier()
    pltpu.sync_copy(acc_sh.at[my_rows], stripe_v)

    def pack_body(r, carry):
        for a in range(8):
            pack_v[r, pl.ds(a * D_EDGE, D_EDGE)] = stripe_v[r * 8 + a, :]
        return carry

    lax.fori_loop(0, PACK_PER_TILE, pack_body, 0)
    pltpu.sync_copy(
        pack_v,
        out_hbm.at[pl.ds(c * (ACC_ROWS // 8) + s * PACK_PER_TILE,
                         PACK_PER_TILE)],
    )


BLK = 1024


def _tc_mlp_body(x_ref, p0_ref, p1_ref, w1a_ref, w1b_ref, b1_ref, w2_ref,
                 b2_ref, g_ref, bt_ref, o_ref):
    x = x_ref[...]
    pp = p0_ref[...] + p1_ref[...]
    wrep = jnp.concatenate([jnp.concatenate([w1b_ref[...]] * 8, axis=0)] * 8,
                           axis=1)
    krow = jax.lax.broadcasted_iota(jnp.int32, (D_FEAT, 8 * D_FEAT), 0)
    kcol = jax.lax.broadcasted_iota(jnp.int32, (D_FEAT, 8 * D_FEAT), 1)
    wbig = jnp.where((krow // D_EDGE) == (kcol // D_FEAT), wrep, 0.0)
    hagg = jnp.dot(pp, wbig, preferred_element_type=jnp.float32)
    hagg = hagg.reshape(BLK // 8, 8, D_FEAT).reshape(BLK, D_FEAT)
    h = jnp.dot(x, w1a_ref[...], preferred_element_type=jnp.float32)
    h = h + hagg + b1_ref[...]
    h = h * jax.nn.sigmoid(h)
    h = jnp.dot(h, w2_ref[...], preferred_element_type=jnp.float32) + b2_ref[...]
    mu = jnp.mean(h, axis=-1, keepdims=True)
    d = h - mu
    var = jnp.mean(d * d, axis=-1, keepdims=True)
    hn = d * lax.rsqrt(var + 1e-5)
    o_ref[...] = x + hn * g_ref[...] + bt_ref[...]


def _full_spec(nr, nc):
    return pl.BlockSpec((nr, nc), lambda i: (0, 0))


_tc_mlp = pl.pallas_call(
    _tc_mlp_body,
    grid=((N_NODES + BLK - 1) // BLK,),
    in_specs=[
        pl.BlockSpec((BLK, D_FEAT), lambda i: (i, 0)),
        pl.BlockSpec((BLK // 8, D_FEAT), lambda i: (i, 0)),
        pl.BlockSpec((BLK // 8, D_FEAT),
                     lambda i: (i + ACC_ROWS // BLK, 0)),
        _full_spec(D_FEAT, D_FEAT),
        _full_spec(D_EDGE, D_FEAT),
        _full_spec(1, D_FEAT),
        _full_spec(D_FEAT, D_FEAT),
        _full_spec(1, D_FEAT),
        _full_spec(1, D_FEAT),
        _full_spec(1, D_FEAT),
    ],
    out_specs=pl.BlockSpec((BLK, D_FEAT), lambda i: (i, 0)),
    out_shape=jax.ShapeDtypeStruct((N_NODES, D_FEAT), jnp.float32),
)


def kernel(x, edge_index, edge_attr, W1, b1, W2, b2, ln_gamma, ln_beta):
    idx3 = edge_index.astype(jnp.int32).reshape(2, NB, BATCH).transpose(1, 0, 2)
    z = edge_attr.T.reshape(2, 8, NB, BATCH).transpose(0, 2, 1, 3)
    zeros = jnp.zeros((ROWS_PER_TILE, D_EDGE), jnp.float32)
    partial = _sc_segment_sum(idx3, z, zeros)
    return _tc_mlp(
        x, partial, partial,
        W1[:D_FEAT], W1[D_FEAT:],
        b1.reshape(1, D_FEAT), W2, b2.reshape(1, D_FEAT),
        ln_gamma.reshape(1, D_FEAT), ln_beta.reshape(1, D_FEAT),
    )
